# baseline (speedup 1.0000x reference)
import functools
import jax
import jax.numpy as jnp
from jax import lax
from jax.experimental import pallas as pl
from jax.experimental.pallas import tpu as pltpu, tpu_sc as plsc

_B, _S, _H = 4, 4096, 1024
_NC, _NS = 2, 16
_NW = _NC * _NS
_ROWS_W = _S // _NW
_R = 16
_CHUNK = _R * _H
_NCHUNK = _ROWS_W // _R


def _sc_body(x_hbm, pos_hbm, out_hbm, posv, xv0, xv1, xv2, xv3):
    wid = lax.axis_index("c") * _NS + lax.axis_index("s")
    xvs = (xv0, xv1, xv2, xv3)

    def chunk_body(c, carry):
        sbase = (wid * _ROWS_W + c * _R) * _H
        pltpu.sync_copy(pos_hbm.at[pl.ds(sbase, _CHUNK)], posv)
        for b in range(_B):
            pltpu.sync_copy(x_hbm.at[pl.ds(b * _S * _H + sbase, _CHUNK)], xvs[b])

        def add_body(i, acc):
            off = i * 16
            pv = posv[pl.ds(off, 16)]
            for b in range(_B):
                xvs[b][pl.ds(off, 16)] += pv
            return acc

        lax.fori_loop(0, _CHUNK // 16, add_body, 0, unroll=4)
        for b in range(_B):
            pltpu.sync_copy(xvs[b], out_hbm.at[pl.ds(b * _S * _H + sbase, _CHUNK)])
        return carry

    lax.fori_loop(0, _NCHUNK, chunk_body, 0)


def kernel(x, position_embeddings):
    B, S, H = x.shape
    xf = x.reshape(-1)
    pf = position_embeddings[:S].reshape(-1)
    mesh = plsc.VectorSubcoreMesh(core_axis_name="c", subcore_axis_name="s")
    run = functools.partial(
        pl.kernel,
        mesh=mesh,
        out_type=jax.ShapeDtypeStruct((B * S * H,), x.dtype),
        scratch_types=[pltpu.VMEM((_CHUNK,), jnp.float32) for _ in range(5)],
    )(_sc_body)
    return run(xf, pf).reshape(B, S, H)


# speedup vs baseline: 1.7985x; 1.7985x over previous
import functools
import jax
import jax.numpy as jnp
from jax import lax
from jax.experimental import pallas as pl
from jax.experimental.pallas import tpu as pltpu, tpu_sc as plsc

_B, _S, _H = 4, 4096, 1024
_NC, _NS = 2, 16
_NW = _NC * _NS
_ROWS_W = _S // _NW
_R = 8
_CH = _R * _H
_NCH = _ROWS_W // _R


def _sc_body(x_hbm, pos_hbm, out_hbm,
             pos0, pos1, xa0, xb0, xc0, xd0, xa1, xb1, xc1, xd1,
             sin0, sin1, sout0, sout1):
    wid = lax.axis_index("c") * _NS + lax.axis_index("s")
    posv = (pos0, pos1)
    xv = ((xa0, xb0, xc0, xd0), (xa1, xb1, xc1, xd1))
    sin = (sin0, sin1)
    sout = (sout0, sout1)

    def sbase(g):
        return (wid * _ROWS_W + g * _R) * _H

    def issue_in(g, p):
        pltpu.async_copy(pos_hbm.at[pl.ds(sbase(g), _CH)], posv[p], sin[p])
        for b in range(_B):
            pltpu.async_copy(x_hbm.at[pl.ds(b * _S * _H + sbase(g), _CH)],
                             xv[p][b], sin[p])

    def drain_in(p):
        pltpu.make_async_copy(pos_hbm.at[pl.ds(0, _CH)], posv[p], sin[p]).wait()
        for b in range(_B):
            pltpu.make_async_copy(x_hbm.at[pl.ds(0, _CH)], xv[p][b], sin[p]).wait()

    def issue_out(g, p):
        for b in range(_B):
            pltpu.async_copy(xv[p][b], out_hbm.at[pl.ds(b * _S * _H + sbase(g), _CH)],
                             sout[p])

    def drain_out(p):
        for b in range(_B):
            pltpu.make_async_copy(xv[p][b], out_hbm.at[pl.ds(0, _CH)], sout[p]).wait()

    def compute(p):
        bufs = xv[p]
        pv_ref = posv[p]

        def add_body(i, acc):
            off = i * 16
            pv = pv_ref[pl.ds(off, 16)]
            for b in range(_B):
                plsc.addupdate(bufs[b].at[pl.ds(off, 16)], pv)
            return acc

        lax.fori_loop(0, _CH // 16, add_body, 0, unroll=8)

    issue_in(0, 0)
    for g in range(_NCH):
        p = g % 2
        if g + 1 < _NCH:
            if g >= 1:
                drain_out(1 - p)
            issue_in(g + 1, 1 - p)
        drain_in(p)
        compute(p)
        issue_out(g, p)
    drain_out(0)
    drain_out(1)


def kernel(x, position_embeddings):
    B, S, H = x.shape
    xf = x.reshape(-1)
    pf = position_embeddings[:S].reshape(-1)
    mesh = plsc.VectorSubcoreMesh(core_axis_name="c", subcore_axis_name="s")
    run = functools.partial(
        pl.kernel,
        mesh=mesh,
        out_type=jax.ShapeDtypeStruct((B * S * H,), x.dtype),
        scratch_types=(
            [pltpu.VMEM((_CH,), jnp.float32) for _ in range(10)]
            + [pltpu.SemaphoreType.DMA for _ in range(4)]
        ),
    )(_sc_body)
    return run(xf, pf).reshape(B, S, H)


# speedup vs baseline: 4.9310x; 2.7417x over previous
import functools
import jax
import jax.numpy as jnp
from jax import lax
from jax.experimental import pallas as pl
from jax.experimental.pallas import tpu as pltpu, tpu_sc as plsc

_B, _S, _H = 4, 4096, 1024
_NC, _NS = 2, 16
_NW = _NC * _NS
_ROWS_W = _S // _NW
_R = 8
_NCH = _ROWS_W // _R


def _sc_body(x_hbm, pos_hbm, out_hbm,
             pos0, pos1, xa0, xb0, xc0, xd0, xa1, xb1, xc1, xd1,
             sin0, sin1, sout0, sout1):
    wid = lax.axis_index("c") * _NS + lax.axis_index("s")
    posv = (pos0, pos1)
    xv = ((xa0, xb0, xc0, xd0), (xa1, xb1, xc1, xd1))
    sin = (sin0, sin1)
    sout = (sout0, sout1)

    def rbase(g):
        return wid * _ROWS_W + g * _R

    def issue_in(g, p):
        pltpu.async_copy(pos_hbm.at[pl.ds(rbase(g), _R), :], posv[p], sin[p])
        for b in range(_B):
            pltpu.async_copy(x_hbm.at[b, pl.ds(rbase(g), _R), :], xv[p][b], sin[p])

    def drain_in(p):
        pltpu.make_async_copy(pos_hbm.at[pl.ds(0, _R), :], posv[p], sin[p]).wait()
        for b in range(_B):
            pltpu.make_async_copy(x_hbm.at[0, pl.ds(0, _R), :], xv[p][b], sin[p]).wait()

    def issue_out(g, p):
        for b in range(_B):
            pltpu.async_copy(xv[p][b], out_hbm.at[b, pl.ds(rbase(g), _R), :], sout[p])

    def drain_out(p):
        for b in range(_B):
            pltpu.make_async_copy(xv[p][b], out_hbm.at[0, pl.ds(0, _R), :], sout[p]).wait()

    def compute(p):
        bufs = xv[p]
        pv_ref = posv[p]

        def row_body(r, acc):
            def col_body(j, acc2):
                cs = j * 16
                pv = pv_ref[r, pl.ds(cs, 16)]
                for b in range(_B):
                    plsc.addupdate(bufs[b].at[r, pl.ds(cs, 16)], pv)
                return acc2

            return lax.fori_loop(0, _H // 16, col_body, acc, unroll=8)

        lax.fori_loop(0, _R, row_body, 0)

    issue_in(0, 0)
    for g in range(_NCH):
        p = g % 2
        if g + 1 < _NCH:
            if g >= 1:
                drain_out(1 - p)
            issue_in(g + 1, 1 - p)
        drain_in(p)
        compute(p)
        issue_out(g, p)
    drain_out(0)
    drain_out(1)


def kernel(x, position_embeddings):
    B, S, H = x.shape
    pf = position_embeddings[:S]
    mesh = plsc.VectorSubcoreMesh(core_axis_name="c", subcore_axis_name="s")
    run = functools.partial(
        pl.kernel,
        mesh=mesh,
        out_type=jax.ShapeDtypeStruct((B, S, H), x.dtype),
        scratch_types=(
            [pltpu.VMEM((_R, _H), jnp.float32) for _ in range(10)]
            + [pltpu.SemaphoreType.DMA for _ in range(4)]
        ),
    )(_sc_body)
    return run(x, pf)


# speedup vs baseline: 4.9941x; 1.0128x over previous
import functools
import jax
import jax.numpy as jnp
from jax import lax
from jax.experimental import pallas as pl
from jax.experimental.pallas import tpu as pltpu, tpu_sc as plsc

_B, _S, _H = 4, 4096, 1024
_S1 = 2048
_NC, _NS = 2, 16
_NW = _NC * _NS
_ROWS_W = _S1 // _NW
_R = 8
_NCH = _ROWS_W // _R
_TC_BS = 1024


def _sc_body(x_hbm, pos_hbm, out_hbm,
             pos0, pos1, xa0, xb0, xc0, xd0, xa1, xb1, xc1, xd1,
             sin0, sin1, sout0, sout1):
    wid = lax.axis_index("c") * _NS + lax.axis_index("s")
    posv = (pos0, pos1)
    xv = ((xa0, xb0, xc0, xd0), (xa1, xb1, xc1, xd1))
    sin = (sin0, sin1)
    sout = (sout0, sout1)

    def rbase(g):
        return wid * _ROWS_W + g * _R

    def issue_in(g, p):
        pltpu.async_copy(pos_hbm.at[pl.ds(rbase(g), _R), :], posv[p], sin[p])
        for b in range(_B):
            pltpu.async_copy(x_hbm.at[b, pl.ds(rbase(g), _R), :], xv[p][b], sin[p])

    def drain_in(p):
        pltpu.make_async_copy(pos_hbm.at[pl.ds(0, _R), :], posv[p], sin[p]).wait()
        for b in range(_B):
            pltpu.make_async_copy(x_hbm.at[0, pl.ds(0, _R), :], xv[p][b], sin[p]).wait()

    def issue_out(g, p):
        for b in range(_B):
            pltpu.async_copy(xv[p][b], out_hbm.at[b, pl.ds(rbase(g), _R), :], sout[p])

    def drain_out(p):
        for b in range(_B):
            pltpu.make_async_copy(xv[p][b], out_hbm.at[0, pl.ds(0, _R), :], sout[p]).wait()

    def compute(p):
        bufs = xv[p]
        pv_ref = posv[p]

        def row_body(r, acc):
            def col_body(j, acc2):
                cs = j * 16
                pv = pv_ref[r, pl.ds(cs, 16)]
                for b in range(_B):
                    plsc.addupdate(bufs[b].at[r, pl.ds(cs, 16)], pv)
                return acc2

            return lax.fori_loop(0, _H // 16, col_body, acc, unroll=8)

        lax.fori_loop(0, _R, row_body, 0)

    issue_in(0, 0)
    for g in range(_NCH):
        p = g % 2
        if g + 1 < _NCH:
            if g >= 1:
                drain_out(1 - p)
            issue_in(g + 1, 1 - p)
        drain_in(p)
        compute(p)
        issue_out(g, p)
    drain_out(0)
    drain_out(1)


def _tc_body(x_ref, p_ref, sc_ref, o_ref):
    o_ref[...] = x_ref[...] + p_ref[...]


def kernel(x, position_embeddings):
    B, S, H = x.shape
    pf = position_embeddings[:S]

    sc_run = functools.partial(
        pl.kernel,
        mesh=plsc.VectorSubcoreMesh(core_axis_name="c", subcore_axis_name="s"),
        out_type=jax.ShapeDtypeStruct((B, S, H), x.dtype),
        scratch_types=(
            [pltpu.VMEM((_R, _H), jnp.float32) for _ in range(10)]
            + [pltpu.SemaphoreType.DMA for _ in range(4)]
        ),
    )(_sc_body)
    sc_out = sc_run(x, pf)

    blk0 = _S1 // _TC_BS
    nblk = (S - _S1) // _TC_BS
    return pl.pallas_call(
        _tc_body,
        grid=(nblk, B),
        in_specs=[
            pl.BlockSpec((1, _TC_BS, H), lambda i, j: (j, i + blk0, 0)),
            pl.BlockSpec((_TC_BS, H), lambda i, j: (i + blk0, 0)),
            pl.BlockSpec(memory_space=pl.ANY),
        ],
        out_specs=pl.BlockSpec((1, _TC_BS, H), lambda i, j: (j, i + blk0, 0)),
        out_shape=jax.ShapeDtypeStruct((B, S, H), x.dtype),
        input_output_aliases={2: 0},
    )(x, pf, sc_out)
